# Initial kernel scaffold; baseline (speedup 1.0000x reference)
import jax
import jax.numpy as jnp
from jax.experimental import pallas as pl


def kernel(context_idxs, emb_table, W, b):
    raise NotImplementedError("write your pallas kernel here")


# speedup vs baseline: 3.8611x; 3.8611x over previous
import functools

import jax
import jax.numpy as jnp
from jax import lax
from jax.experimental import pallas as pl
from jax.experimental.pallas import tpu as pltpu
from jax.experimental.pallas import tpu_sc as plsc

VOCAB = 100000
D = 128
B = 1024
L = 200

NC = 2
NS = 16
NW = NC * NS
LPW = -(-L // NW)

CHUNK = 128
NCHUNK = B // CHUNK


def _pool_body(table_hbm, idx_hbm, out_hbm, idx_v, rows_v, acc_v, sem):
    wid = lax.axis_index("c") * NS + lax.axis_index("s")
    for j in range(LPW):
        l = wid * LPW + j

        @pl.when(l < L)
        def _():
            pltpu.sync_copy(idx_hbm.at[l], idx_v)
            acc = tuple(jnp.zeros((16,), jnp.float32) for _ in range(8))
            for c in range(NCHUNK):
                pltpu.async_copy(table_hbm.at[idx_v.at[c]], rows_v, sem).wait()

                def rbody(r, a):
                    return tuple(
                        a[k] + rows_v[r, k * 16:(k + 1) * 16] for k in range(8)
                    )

                acc = lax.fori_loop(0, CHUNK, rbody, acc)
            for k in range(8):
                acc_v[k * 16:(k + 1) * 16] = acc[k] * (1.0 / B)
            pltpu.sync_copy(acc_v, out_hbm.at[l])


@jax.jit
def _sc_pool(emb_table, idx3):
    mesh = plsc.VectorSubcoreMesh(core_axis_name="c", subcore_axis_name="s")
    f = pl.kernel(
        _pool_body,
        mesh=mesh,
        out_type=jax.ShapeDtypeStruct((L, D), jnp.float32),
        scratch_types=[
            pltpu.VMEM((NCHUNK, CHUNK), jnp.int32),
            pltpu.VMEM((CHUNK, D), jnp.float32),
            pltpu.VMEM((D,), jnp.float32),
            pltpu.SemaphoreType.DMA,
        ],
    )
    return f(emb_table, idx3)


NBLK = 2048
GRID = -(-VOCAB // NBLK)


def _mm_body(x_ref, w_ref, b_ref, o_ref):
    o_ref[...] = lax.dot_general(
        x_ref[...], w_ref[...],
        (((1,), (1,)), ((), ())),
        preferred_element_type=jnp.float32,
    ) + b_ref[...]


@jax.jit
def _tc_project(pooled, W, b2d):
    return pl.pallas_call(
        _mm_body,
        grid=(GRID,),
        in_specs=[
            pl.BlockSpec((L, D), lambda i: (0, 0)),
            pl.BlockSpec((NBLK, D), lambda i: (i, 0)),
            pl.BlockSpec((1, NBLK), lambda i: (0, i)),
        ],
        out_specs=pl.BlockSpec((L, NBLK), lambda i: (0, i)),
        out_shape=jax.ShapeDtypeStruct((L, VOCAB), jnp.float32),
    )(pooled, W, b2d)


def kernel(context_idxs, emb_table, W, b):
    idx3 = context_idxs.T.reshape(L, NCHUNK, CHUNK).astype(jnp.int32)
    pooled = _sc_pool(emb_table, idx3)
    return _tc_project(pooled, W, b.reshape(1, VOCAB))


# speedup vs baseline: 4.6272x; 1.1984x over previous
import functools

import jax
import jax.numpy as jnp
from jax import lax
from jax.experimental import pallas as pl
from jax.experimental.pallas import tpu as pltpu
from jax.experimental.pallas import tpu_sc as plsc

VOCAB = 100000
D = 128
B = 1024
L = 200

NC = 2
NS = 16
NW = NC * NS
LPW = -(-L // NW)

CHUNK = 128
NCHUNK = B // CHUNK


def _pool_body(table_hbm, idx_hbm, out_hbm, idx_v, rows_v, acc_v, sem0, sem1):
    wid = lax.axis_index("c") * NS + lax.axis_index("s")
    sems = (sem0, sem1)
    for j in range(LPW):
        l = wid * LPW + j

        @pl.when(l < L)
        def _():
            pltpu.sync_copy(idx_hbm.at[l], idx_v)
            acc = tuple(jnp.zeros((16,), jnp.float32) for _ in range(8))
            cps = [None] * NCHUNK
            cps[0] = pltpu.async_copy(
                table_hbm.at[idx_v.at[0]], rows_v.at[0], sems[0])
            for c in range(NCHUNK):
                if c + 1 < NCHUNK:
                    nb = (c + 1) % 2
                    cps[c + 1] = pltpu.async_copy(
                        table_hbm.at[idx_v.at[c + 1]], rows_v.at[nb], sems[nb])
                cps[c].wait()
                buf = c % 2

                def rbody(r, a):
                    return tuple(
                        a[k] + rows_v[buf, r, k * 16:(k + 1) * 16]
                        for k in range(8)
                    )

                acc = lax.fori_loop(0, CHUNK, rbody, acc, unroll=8)
            for k in range(8):
                acc_v[k * 16:(k + 1) * 16] = acc[k] * (1.0 / B)
            pltpu.sync_copy(acc_v, out_hbm.at[l])


@jax.jit
def _sc_pool(emb_table, idx3):
    mesh = plsc.VectorSubcoreMesh(core_axis_name="c", subcore_axis_name="s")
    f = pl.kernel(
        _pool_body,
        mesh=mesh,
        out_type=jax.ShapeDtypeStruct((L, D), jnp.float32),
        scratch_types=[
            pltpu.VMEM((NCHUNK, CHUNK), jnp.int32),
            pltpu.VMEM((2, CHUNK, D), jnp.float32),
            pltpu.VMEM((D,), jnp.float32),
            pltpu.SemaphoreType.DMA,
            pltpu.SemaphoreType.DMA,
        ],
    )
    return f(emb_table, idx3)


NBLK = 2048
GRID = -(-VOCAB // NBLK)


def _mm_body(x_ref, w_ref, b_ref, o_ref):
    x = x_ref[...].astype(jnp.bfloat16)
    w = w_ref[...].astype(jnp.bfloat16)
    o_ref[...] = lax.dot_general(
        x, w,
        (((1,), (1,)), ((), ())),
        preferred_element_type=jnp.float32,
    ) + b_ref[...]


@jax.jit
def _tc_project(pooled, W, b2d):
    return pl.pallas_call(
        _mm_body,
        grid=(GRID,),
        in_specs=[
            pl.BlockSpec((L, D), lambda i: (0, 0)),
            pl.BlockSpec((NBLK, D), lambda i: (i, 0)),
            pl.BlockSpec((1, NBLK), lambda i: (0, i)),
        ],
        out_specs=pl.BlockSpec((L, NBLK), lambda i: (0, i)),
        out_shape=jax.ShapeDtypeStruct((L, VOCAB), jnp.float32),
    )(pooled, W, b2d)


def kernel(context_idxs, emb_table, W, b):
    idx3 = context_idxs.T.reshape(L, NCHUNK, CHUNK).astype(jnp.int32)
    pooled = _sc_pool(emb_table, idx3)
    return _tc_project(pooled, W, b.reshape(1, VOCAB))


# speedup vs baseline: 5.1248x; 1.1075x over previous
import functools

import jax
import jax.numpy as jnp
from jax import lax
from jax.experimental import pallas as pl
from jax.experimental.pallas import tpu as pltpu
from jax.experimental.pallas import tpu_sc as plsc

VOCAB = 100000
D = 128
B = 1024
L = 200

NC = 2
NS = 16
NW = NC * NS
LPW = -(-L // NW)

CHUNK = 128
NCHUNK = B // CHUNK


def _pool_body(table_hbm, idx_hbm, out_hbm, idx_v, rows_v, acc_v, sem0, sem1):
    wid = lax.axis_index("c") * NS + lax.axis_index("s")
    sems = (sem0, sem1)
    for j in range(LPW):
        l = wid * LPW + j

        @pl.when(l < L)
        def _():
            pltpu.sync_copy(idx_hbm.at[l], idx_v)
            acc = tuple(jnp.zeros((16,), jnp.float32) for _ in range(8))
            cps = [None] * NCHUNK
            cps[0] = pltpu.async_copy(
                table_hbm.at[idx_v.at[0]], rows_v.at[0], sems[0])
            for c in range(NCHUNK):
                if c + 1 < NCHUNK:
                    nb = (c + 1) % 2
                    cps[c + 1] = pltpu.async_copy(
                        table_hbm.at[idx_v.at[c + 1]], rows_v.at[nb], sems[nb])
                cps[c].wait()
                buf = c % 2

                def rbody(r, a):
                    return tuple(
                        a[k] + rows_v[buf, r, k * 16:(k + 1) * 16]
                        for k in range(8)
                    )

                acc = lax.fori_loop(0, CHUNK, rbody, acc, unroll=8)
            for k in range(8):
                acc_v[k * 16:(k + 1) * 16] = acc[k] * (1.0 / B)
            pltpu.sync_copy(acc_v, out_hbm.at[l])


@jax.jit
def _sc_pool(emb_table, idx3):
    mesh = plsc.VectorSubcoreMesh(core_axis_name="c", subcore_axis_name="s")
    f = pl.kernel(
        _pool_body,
        mesh=mesh,
        out_type=jax.ShapeDtypeStruct((L, D), jnp.float32),
        scratch_types=[
            pltpu.VMEM((NCHUNK, CHUNK), jnp.int32),
            pltpu.VMEM((2, CHUNK, D), jnp.float32),
            pltpu.VMEM((D,), jnp.float32),
            pltpu.SemaphoreType.DMA,
            pltpu.SemaphoreType.DMA,
        ],
    )
    return f(emb_table, idx3)


NBLK = 4096
GRID = -(-VOCAB // NBLK)


def _mm_body(x_ref, w_ref, b_ref, o_ref):
    x = x_ref[...].astype(jnp.bfloat16)
    w = w_ref[...].astype(jnp.bfloat16)
    o_ref[...] = lax.dot_general(
        x, w,
        (((1,), (1,)), ((), ())),
        preferred_element_type=jnp.float32,
    ) + b_ref[...]


@jax.jit
def _tc_project(pooled, W, b2d):
    return pl.pallas_call(
        _mm_body,
        grid=(GRID,),
        in_specs=[
            pl.BlockSpec((L, D), lambda i: (0, 0)),
            pl.BlockSpec((NBLK, D), lambda i: (i, 0)),
            pl.BlockSpec((1, NBLK), lambda i: (0, i)),
        ],
        out_specs=pl.BlockSpec((L, NBLK), lambda i: (0, i)),
        out_shape=jax.ShapeDtypeStruct((L, VOCAB), jnp.float32),
    )(pooled, W, b2d)


def kernel(context_idxs, emb_table, W, b):
    idx3 = context_idxs.T.reshape(L, NCHUNK, CHUNK).astype(jnp.int32)
    pooled = _sc_pool(emb_table, idx3)
    return _tc_project(pooled, W, b.reshape(1, VOCAB))


# speedup vs baseline: 5.2492x; 1.0243x over previous
import functools

import jax
import jax.numpy as jnp
from jax import lax
from jax.experimental import pallas as pl
from jax.experimental.pallas import tpu as pltpu
from jax.experimental.pallas import tpu_sc as plsc

VOCAB = 100000
D = 128
B = 1024
L = 200

NC = 2
NS = 16
NW = NC * NS
LPW = -(-L // NW)

CHUNK = 128
NCHUNK = B // CHUNK


def _pool_body(table_hbm, idx_hbm, out_hbm, idx_v, rows_v, acc_v, sem0, sem1):
    wid = lax.axis_index("c") * NS + lax.axis_index("s")
    sems = (sem0, sem1)
    for j in range(LPW):
        l = wid * LPW + j

        @pl.when(l < L)
        def _():
            pltpu.sync_copy(idx_hbm.at[l], idx_v)
            acc = tuple(jnp.zeros((16,), jnp.float32) for _ in range(8))
            cps = [None] * NCHUNK
            cps[0] = pltpu.async_copy(
                table_hbm.at[idx_v.at[0]], rows_v.at[0], sems[0])
            for c in range(NCHUNK):
                if c + 1 < NCHUNK:
                    nb = (c + 1) % 2
                    cps[c + 1] = pltpu.async_copy(
                        table_hbm.at[idx_v.at[c + 1]], rows_v.at[nb], sems[nb])
                cps[c].wait()
                buf = c % 2

                def rbody(r, a):
                    return tuple(
                        a[k] + rows_v[buf, r, k * 16:(k + 1) * 16]
                        for k in range(8)
                    )

                acc = lax.fori_loop(0, CHUNK, rbody, acc, unroll=8)
            for k in range(8):
                acc_v[k * 16:(k + 1) * 16] = acc[k] * (1.0 / B)
            pltpu.sync_copy(acc_v, out_hbm.at[l])


@jax.jit
def _sc_pool(emb_table, idx3):
    mesh = plsc.VectorSubcoreMesh(core_axis_name="c", subcore_axis_name="s")
    f = pl.kernel(
        _pool_body,
        mesh=mesh,
        out_type=jax.ShapeDtypeStruct((L, D), jnp.float32),
        scratch_types=[
            pltpu.VMEM((NCHUNK, CHUNK), jnp.int32),
            pltpu.VMEM((2, CHUNK, D), jnp.float32),
            pltpu.VMEM((D,), jnp.float32),
            pltpu.SemaphoreType.DMA,
            pltpu.SemaphoreType.DMA,
        ],
    )
    return f(emb_table, idx3)


NBLK = 8192
GRID = -(-VOCAB // NBLK)


def _mm_body(x_ref, w_ref, b_ref, o_ref):
    x = x_ref[...].astype(jnp.bfloat16)
    w = w_ref[...].astype(jnp.bfloat16)
    o_ref[...] = lax.dot_general(
        x, w,
        (((1,), (1,)), ((), ())),
        preferred_element_type=jnp.float32,
    ) + b_ref[...]


@jax.jit
def _tc_project(pooled, W, b2d):
    return pl.pallas_call(
        _mm_body,
        grid=(GRID,),
        in_specs=[
            pl.BlockSpec((L, D), lambda i: (0, 0)),
            pl.BlockSpec((NBLK, D), lambda i: (i, 0)),
            pl.BlockSpec((1, NBLK), lambda i: (0, i)),
        ],
        out_specs=pl.BlockSpec((L, NBLK), lambda i: (0, i)),
        out_shape=jax.ShapeDtypeStruct((L, VOCAB), jnp.float32),
    )(pooled, W, b2d)


def kernel(context_idxs, emb_table, W, b):
    idx3 = context_idxs.T.reshape(L, NCHUNK, CHUNK).astype(jnp.int32)
    pooled = _sc_pool(emb_table, idx3)
    return _tc_project(pooled, W, b.reshape(1, VOCAB))


# speedup vs baseline: 5.3181x; 1.0131x over previous
import functools

import jax
import jax.numpy as jnp
from jax import lax
from jax.experimental import pallas as pl
from jax.experimental.pallas import tpu as pltpu
from jax.experimental.pallas import tpu_sc as plsc

VOCAB = 100000
D = 128
B = 1024
L = 200

NC = 2
NS = 16
NW = NC * NS
LPW = -(-L // NW)

CHUNK = 128
NCHUNK = B // CHUNK


def _pool_body(table_hbm, idx_hbm, out_hbm, idx_v, rows_v, acc_v, sem0, sem1):
    wid = lax.axis_index("c") * NS + lax.axis_index("s")
    sems = (sem0, sem1)
    for j in range(LPW):
        l = wid * LPW + j

        @pl.when(l < L)
        def _():
            pltpu.sync_copy(idx_hbm.at[l], idx_v)
            acc = tuple(jnp.zeros((16,), jnp.float32) for _ in range(8))
            cps = [None] * NCHUNK
            cps[0] = pltpu.async_copy(
                table_hbm.at[idx_v.at[0]], rows_v.at[0], sems[0])
            for c in range(NCHUNK):
                if c + 1 < NCHUNK:
                    nb = (c + 1) % 2
                    cps[c + 1] = pltpu.async_copy(
                        table_hbm.at[idx_v.at[c + 1]], rows_v.at[nb], sems[nb])
                cps[c].wait()
                buf = c % 2

                def rbody(r, a):
                    return tuple(
                        a[k] + rows_v[buf, r, k * 16:(k + 1) * 16]
                        for k in range(8)
                    )

                acc = lax.fori_loop(0, CHUNK, rbody, acc, unroll=8)
            for k in range(8):
                acc_v[k * 16:(k + 1) * 16] = acc[k] * (1.0 / B)
            pltpu.sync_copy(acc_v, out_hbm.at[l])


@jax.jit
def _sc_pool(emb_table, idx3):
    mesh = plsc.VectorSubcoreMesh(core_axis_name="c", subcore_axis_name="s")
    f = pl.kernel(
        _pool_body,
        mesh=mesh,
        out_type=jax.ShapeDtypeStruct((L, D), jnp.float32),
        scratch_types=[
            pltpu.VMEM((NCHUNK, CHUNK), jnp.int32),
            pltpu.VMEM((2, CHUNK, D), jnp.float32),
            pltpu.VMEM((D,), jnp.float32),
            pltpu.SemaphoreType.DMA,
            pltpu.SemaphoreType.DMA,
        ],
    )
    return f(emb_table, idx3)


NBLK = 16384
GRID = -(-VOCAB // NBLK)


def _mm_body(x_ref, w_ref, b_ref, o_ref):
    x = x_ref[...].astype(jnp.bfloat16)
    w = w_ref[...].astype(jnp.bfloat16)
    o_ref[...] = lax.dot_general(
        x, w,
        (((1,), (1,)), ((), ())),
        preferred_element_type=jnp.float32,
    ) + b_ref[...]


@jax.jit
def _tc_project(pooled, W, b2d):
    return pl.pallas_call(
        _mm_body,
        grid=(GRID,),
        in_specs=[
            pl.BlockSpec((L, D), lambda i: (0, 0)),
            pl.BlockSpec((NBLK, D), lambda i: (i, 0)),
            pl.BlockSpec((1, NBLK), lambda i: (0, i)),
        ],
        out_specs=pl.BlockSpec((L, NBLK), lambda i: (0, i)),
        out_shape=jax.ShapeDtypeStruct((L, VOCAB), jnp.float32),
    )(pooled, W, b2d)


def kernel(context_idxs, emb_table, W, b):
    idx3 = context_idxs.T.reshape(L, NCHUNK, CHUNK).astype(jnp.int32)
    pooled = _sc_pool(emb_table, idx3)
    return _tc_project(pooled, W, b.reshape(1, VOCAB))


# speedup vs baseline: 5.7164x; 1.0749x over previous
import functools

import jax
import jax.numpy as jnp
from jax import lax
from jax.experimental import pallas as pl
from jax.experimental.pallas import tpu as pltpu
from jax.experimental.pallas import tpu_sc as plsc

VOCAB = 100000
D = 128
B = 1024
L = 200

NC = 2
NS = 16
NW = NC * NS
LPW = -(-L // NW)

CHUNK = 128
NCHUNK = B // CHUNK


NHEAVY = L - NW * (L // NW)
LO = L // NW
NRING = 4


def _pool_body(table_hbm, idx_hbm, out_hbm, idx_v, rows_v, acc_v,
               isem, sem0, sem1, sem2, sem3):
    wid = lax.axis_index("s") * NC + lax.axis_index("c")
    start = jnp.where(wid < NHEAVY, wid * (LO + 1),
                      NHEAVY * (LO + 1) + (wid - NHEAVY) * LO)
    n = jnp.where(wid < NHEAVY, LO + 1, LO)
    sems = (sem0, sem1, sem2, sem3)

    pltpu.sync_copy(idx_hbm.at[start], idx_v.at[0])
    for j in range(LPW):
        l = start + j

        @pl.when(j < n)
        def _():
            ib = j % 2
            if j + 1 < LPW:
                @pl.when(j + 1 < n)
                def _():
                    pltpu.async_copy(
                        idx_hbm.at[l + 1], idx_v.at[(j + 1) % 2], isem)
            acc = tuple(jnp.zeros((16,), jnp.float32) for _ in range(8))
            cps = [None] * NCHUNK
            for c in range(NRING - 1):
                cps[c] = pltpu.async_copy(
                    table_hbm.at[idx_v.at[ib, c]], rows_v.at[c % NRING],
                    sems[c % NRING])
            for c in range(NCHUNK):
                if c + NRING - 1 < NCHUNK:
                    nb = (c + NRING - 1) % NRING
                    cps[c + NRING - 1] = pltpu.async_copy(
                        table_hbm.at[idx_v.at[ib, c + NRING - 1]],
                        rows_v.at[nb], sems[nb])
                cps[c].wait()
                buf = c % NRING

                def rbody(r, a):
                    return tuple(
                        a[k] + rows_v[buf, r, k * 16:(k + 1) * 16]
                        for k in range(8)
                    )

                acc = lax.fori_loop(0, CHUNK, rbody, acc, unroll=8)
            for k in range(8):
                acc_v[k * 16:(k + 1) * 16] = acc[k] * (1.0 / B)
            pltpu.sync_copy(acc_v, out_hbm.at[l])
            if j + 1 < LPW:
                @pl.when(j + 1 < n)
                def _():
                    pltpu.make_async_copy(
                        idx_hbm.at[l + 1], idx_v.at[(j + 1) % 2], isem).wait()


@jax.jit
def _sc_pool(emb_table, idx3):
    mesh = plsc.VectorSubcoreMesh(core_axis_name="c", subcore_axis_name="s")
    f = pl.kernel(
        _pool_body,
        mesh=mesh,
        out_type=jax.ShapeDtypeStruct((L, D), jnp.float32),
        scratch_types=[
            pltpu.VMEM((2, NCHUNK, CHUNK), jnp.int32),
            pltpu.VMEM((NRING, CHUNK, D), jnp.float32),
            pltpu.VMEM((D,), jnp.float32),
            pltpu.SemaphoreType.DMA,
            pltpu.SemaphoreType.DMA,
            pltpu.SemaphoreType.DMA,
            pltpu.SemaphoreType.DMA,
            pltpu.SemaphoreType.DMA,
        ],
    )
    return f(emb_table, idx3)


NBLK = 16384
GRID = -(-VOCAB // NBLK)


def _mm_body(x_ref, w_ref, b_ref, o_ref):
    x = x_ref[...].astype(jnp.bfloat16)
    w = w_ref[...].astype(jnp.bfloat16)
    o_ref[...] = lax.dot_general(
        x, w,
        (((1,), (1,)), ((), ())),
        preferred_element_type=jnp.float32,
    ) + b_ref[...]


@jax.jit
def _tc_project(pooled, W, b2d):
    return pl.pallas_call(
        _mm_body,
        grid=(GRID,),
        in_specs=[
            pl.BlockSpec((L, D), lambda i: (0, 0)),
            pl.BlockSpec((NBLK, D), lambda i: (i, 0)),
            pl.BlockSpec((1, NBLK), lambda i: (0, i)),
        ],
        out_specs=pl.BlockSpec((L, NBLK), lambda i: (0, i)),
        out_shape=jax.ShapeDtypeStruct((L, VOCAB), jnp.float32),
    )(pooled, W, b2d)


def kernel(context_idxs, emb_table, W, b):
    idx3 = context_idxs.T.reshape(L, NCHUNK, CHUNK).astype(jnp.int32)
    pooled = _sc_pool(emb_table, idx3)
    return _tc_project(pooled, W, b.reshape(1, VOCAB))
